# Initial kernel scaffold; baseline (speedup 1.0000x reference)
import functools

import jax
import jax.numpy as jnp
from jax import lax
from jax.experimental import pallas as pl
from jax.experimental.pallas import tpu as pltpu
from jax.experimental.pallas import tpu_sc as plsc

N = 10000
E = 320000
D = 128
NC = 2
NS = 16
NW = NC * NS
CH = 128
NCHUNK = 79
EPW = NCHUNK * CH
CAP = NW * EPW
NP = 10016
STRIPE = NP // NS

_MESH = dict(core_axis_name="c", subcore_axis_name="s")


def _sc_degree(dst3, ones_rows, zeros16):
    mesh = plsc.VectorSubcoreMesh(**_MESH)

    @functools.partial(
        pl.kernel,
        out_type=jax.ShapeDtypeStruct((NC, NP, 16), jnp.float32),
        mesh=mesh,
        scratch_types=[
            pltpu.VMEM((NCHUNK, CH), jnp.int32),
            pltpu.VMEM((CH, 16), jnp.float32),
            pltpu.VMEM_SHARED((NP, 16), jnp.float32),
        ],
    )
    def k(dst_hbm, ones_hbm, zero_hbm, out_hbm, dst_v, ones_v, acc):
        c = lax.axis_index("c")
        s = lax.axis_index("s")
        wid = c * NS + s
        base = s * STRIPE
        pltpu.sync_copy(zero_hbm, acc.at[pl.ds(base, STRIPE)])
        pltpu.sync_copy(dst_hbm.at[wid], dst_v)
        pltpu.sync_copy(ones_hbm, ones_v)
        plsc.subcore_barrier()

        @pl.loop(0, NCHUNK)
        def _(j):
            pltpu.sync_copy(ones_v, acc.at[dst_v.at[j]], add=True)

        plsc.subcore_barrier()
        pltpu.sync_copy(acc.at[pl.ds(base, STRIPE)],
                        out_hbm.at[c, pl.ds(base, STRIPE)])

    return k(dst3, ones_rows, zeros16)


def _sc_spmm(hs, src3, dst3, zeros128):
    mesh = plsc.VectorSubcoreMesh(**_MESH)

    @functools.partial(
        pl.kernel,
        out_type=jax.ShapeDtypeStruct((NC, NP, D), jnp.float32),
        mesh=mesh,
        scratch_types=[
            pltpu.VMEM((NCHUNK, CH), jnp.int32),
            pltpu.VMEM((NCHUNK, CH), jnp.int32),
            pltpu.VMEM((CH, D), jnp.float32),
            pltpu.VMEM_SHARED((NP, D), jnp.float32),
        ],
    )
    def k(hs_hbm, src_hbm, dst_hbm, zero_hbm, out_hbm, src_v, dst_v, rows_v, acc):
        c = lax.axis_index("c")
        s = lax.axis_index("s")
        wid = c * NS + s
        base = s * STRIPE
        pltpu.sync_copy(zero_hbm, acc.at[pl.ds(base, STRIPE)])
        pltpu.sync_copy(src_hbm.at[wid], src_v)
        pltpu.sync_copy(dst_hbm.at[wid], dst_v)
        plsc.subcore_barrier()

        @pl.loop(0, NCHUNK)
        def _(j):
            pltpu.sync_copy(hs_hbm.at[src_v.at[j]], rows_v)
            pltpu.sync_copy(rows_v, acc.at[dst_v.at[j]], add=True)

        plsc.subcore_barrier()
        pltpu.sync_copy(acc.at[pl.ds(base, STRIPE)],
                        out_hbm.at[c, pl.ds(base, STRIPE)])

    return k(hs, src3, dst3, zeros128)


_PREC = lax.Precision.HIGHEST


def _tc_pre(xp, W1, degp):
    def body(x_ref, w_ref, deg_ref, hs_ref, dis_ref):
        deg = deg_ref[0, :, 0:1] + deg_ref[1, :, 0:1] + 1.0
        dis = lax.rsqrt(deg)
        h = jnp.dot(x_ref[...], w_ref[...],
                    preferred_element_type=jnp.float32, precision=_PREC)
        hs_ref[...] = h * dis
        dis_ref[...] = dis

    return pl.pallas_call(
        body,
        out_shape=(
            jax.ShapeDtypeStruct((NP, D), jnp.float32),
            jax.ShapeDtypeStruct((NP, 1), jnp.float32),
        ),
    )(xp, W1, degp)


def _tc_mid(s1, hs1, dis, b1, W2):
    def body(s_ref, hs_ref, dis_ref, b_ref, w_ref, out_ref):
        dis = dis_ref[...]
        h = dis * (s_ref[0] + s_ref[1] + hs_ref[...]) + b_ref[...]
        h = jnp.maximum(h, 0.0)
        rows = lax.broadcasted_iota(jnp.int32, (NP, 1), 0)
        h = jnp.where(rows < N, h, 0.0)
        out_ref[...] = dis * jnp.dot(h, w_ref[...],
                                     preferred_element_type=jnp.float32,
                                     precision=_PREC)

    return pl.pallas_call(
        body,
        out_shape=jax.ShapeDtypeStruct((NP, D), jnp.float32),
    )(s1, hs1, dis, b1, W2)


def _tc_post(s2, hs2, dis, b2, Wfc, bfc):
    def body(s_ref, hs_ref, dis_ref, b_ref, w_ref, bf_ref, out_ref):
        dis = dis_ref[...]
        h = dis * (s_ref[0] + s_ref[1] + hs_ref[...]) + b_ref[...]
        h = jnp.maximum(h, 0.0)
        out_ref[...] = jnp.dot(h, w_ref[...],
                               preferred_element_type=jnp.float32,
                               precision=_PREC) + bf_ref[...]

    return pl.pallas_call(
        body,
        out_shape=jax.ShapeDtypeStruct((NP, 1), jnp.float32),
    )(s2, hs2, dis, b2, Wfc, bfc)


def kernel(x, edge_index, W1, b1, W2, b2, Wfc, bfc):
    src = edge_index[0].astype(jnp.int32)
    dst = edge_index[1].astype(jnp.int32)
    pad_idx = jnp.full((CAP - E,), N, jnp.int32)
    src3 = jnp.concatenate([src, pad_idx]).reshape(NW, NCHUNK, CH)
    dst3 = jnp.concatenate([dst, pad_idx]).reshape(NW, NCHUNK, CH)

    xp = jnp.zeros((NP, D), jnp.float32).at[:N].set(x)
    ones_rows = jnp.ones((CH, 16), jnp.float32)
    zeros16 = jnp.zeros((STRIPE, 16), jnp.float32)
    zeros128 = jnp.zeros((STRIPE, D), jnp.float32)

    degp = _sc_degree(dst3, ones_rows, zeros16)
    hs1, dis = _tc_pre(xp, W1, degp)
    s1 = _sc_spmm(hs1, src3, dst3, zeros128)
    hs2 = _tc_mid(s1, hs1, dis, b1.reshape(1, D), W2)
    s2 = _sc_spmm(hs2, src3, dst3, zeros128)
    outp = _tc_post(s2, hs2, dis, b2.reshape(1, D),
                    Wfc, bfc.reshape(1, 1))
    return outp[:N]


# speedup vs baseline: 13.5472x; 13.5472x over previous
import functools

import jax
import jax.numpy as jnp
from jax import lax
from jax.experimental import pallas as pl
from jax.experimental.pallas import tpu as pltpu
from jax.experimental.pallas import tpu_sc as plsc

N = 10000
E = 320000
D = 128
NC = 2
NS = 16
NW = NC * NS
CH = 128
NCHUNK = 79
EPW = NCHUNK * CH
CAP = NW * EPW
NP = 10112
STRIPE = NP // NS

_MESH = dict(core_axis_name="c", subcore_axis_name="s")


def _sc_degree(dst3, ones_rows, zeros16):
    mesh = plsc.VectorSubcoreMesh(**_MESH)

    @functools.partial(
        pl.kernel,
        out_type=jax.ShapeDtypeStruct((NC, NP, 16), jnp.float32),
        mesh=mesh,
        scratch_types=[
            pltpu.VMEM((NCHUNK, CH), jnp.int32),
            pltpu.VMEM((CH, 16), jnp.float32),
            pltpu.VMEM_SHARED((NP, 16), jnp.float32),
        ],
    )
    def k(dst_hbm, ones_hbm, zero_hbm, out_hbm, dst_v, ones_v, acc):
        c = lax.axis_index("c")
        s = lax.axis_index("s")
        wid = c * NS + s
        base = s * STRIPE
        pltpu.sync_copy(zero_hbm, acc.at[pl.ds(base, STRIPE)])
        pltpu.sync_copy(dst_hbm.at[wid], dst_v)
        pltpu.sync_copy(ones_hbm, ones_v)
        plsc.subcore_barrier()

        @pl.loop(0, NCHUNK)
        def _(j):
            pltpu.sync_copy(ones_v, acc.at[dst_v.at[j]], add=True)

        plsc.subcore_barrier()
        pltpu.sync_copy(acc.at[pl.ds(base, STRIPE)],
                        out_hbm.at[c, pl.ds(base, STRIPE)])

    return k(dst3, ones_rows, zeros16)


def _sc_spmm(hs, src3, dst3, zeros128):
    mesh = plsc.VectorSubcoreMesh(**_MESH)

    @functools.partial(
        pl.kernel,
        out_type=jax.ShapeDtypeStruct((NC, NP, D), jnp.float32),
        mesh=mesh,
        scratch_types=[
            pltpu.VMEM((NCHUNK, CH), jnp.int32),
            pltpu.VMEM((NCHUNK, CH), jnp.int32),
            pltpu.VMEM((CH, D), jnp.float32),
            pltpu.VMEM_SHARED((NP, D), jnp.float32),
        ],
    )
    def k(hs_hbm, src_hbm, dst_hbm, zero_hbm, out_hbm, src_v, dst_v, rows_v, acc):
        c = lax.axis_index("c")
        s = lax.axis_index("s")
        wid = c * NS + s
        base = s * STRIPE
        pltpu.sync_copy(zero_hbm, acc.at[pl.ds(base, STRIPE)])
        pltpu.sync_copy(src_hbm.at[wid], src_v)
        pltpu.sync_copy(dst_hbm.at[wid], dst_v)
        plsc.subcore_barrier()

        @pl.loop(0, NCHUNK)
        def _(j):
            pltpu.sync_copy(hs_hbm.at[src_v.at[j]], rows_v)
            pltpu.sync_copy(rows_v, acc.at[dst_v.at[j]], add=True)

        plsc.subcore_barrier()
        pltpu.sync_copy(acc.at[pl.ds(base, STRIPE)],
                        out_hbm.at[c, pl.ds(base, STRIPE)])

    return k(hs, src3, dst3, zeros128)


_PREC = lax.Precision.HIGHEST


def _tc_pre(xp, W1, degp):
    def body(x_ref, w_ref, deg_ref, hs_ref, dis_ref):
        deg = deg_ref[0, :, 0:1] + deg_ref[1, :, 0:1] + 1.0
        dis = lax.rsqrt(deg)
        h = jnp.dot(x_ref[...], w_ref[...],
                    preferred_element_type=jnp.float32, precision=_PREC)
        hs_ref[...] = h * dis
        dis_ref[...] = dis

    return pl.pallas_call(
        body,
        out_shape=(
            jax.ShapeDtypeStruct((NP, D), jnp.float32),
            jax.ShapeDtypeStruct((NP, 1), jnp.float32),
        ),
    )(xp, W1, degp)


def _tc_mid(s1, hs1, dis, b1, W2):
    def body(s_ref, hs_ref, dis_ref, b_ref, w_ref, out_ref):
        dis = dis_ref[...]
        h = dis * (s_ref[0] + s_ref[1] + hs_ref[...]) + b_ref[...]
        h = jnp.maximum(h, 0.0)
        rows = lax.broadcasted_iota(jnp.int32, (NP, 1), 0)
        h = jnp.where(rows < N, h, 0.0)
        out_ref[...] = dis * jnp.dot(h, w_ref[...],
                                     preferred_element_type=jnp.float32,
                                     precision=_PREC)

    return pl.pallas_call(
        body,
        out_shape=jax.ShapeDtypeStruct((NP, D), jnp.float32),
    )(s1, hs1, dis, b1, W2)


def _tc_post(s2, hs2, dis, b2, Wfc, bfc):
    def body(s_ref, hs_ref, dis_ref, b_ref, w_ref, bf_ref, out_ref):
        dis = dis_ref[...]
        h = dis * (s_ref[0] + s_ref[1] + hs_ref[...]) + b_ref[...]
        h = jnp.maximum(h, 0.0)
        out_ref[...] = jnp.dot(h, w_ref[...],
                               preferred_element_type=jnp.float32,
                               precision=_PREC) + bf_ref[...]

    return pl.pallas_call(
        body,
        out_shape=jax.ShapeDtypeStruct((NP, 1), jnp.float32),
    )(s2, hs2, dis, b2, Wfc, bfc)


def kernel(x, edge_index, W1, b1, W2, b2, Wfc, bfc):
    src = edge_index[0].astype(jnp.int32)
    dst = edge_index[1].astype(jnp.int32)
    pad_idx = jnp.full((CAP - E,), N, jnp.int32)
    src3 = jnp.concatenate([src, pad_idx]).reshape(NW, NCHUNK, CH)
    dst3 = jnp.concatenate([dst, pad_idx]).reshape(NW, NCHUNK, CH)

    xp = jnp.zeros((NP, D), jnp.float32).at[:N].set(x)
    ones_rows = jnp.ones((CH, 16), jnp.float32)
    zeros16 = jnp.zeros((STRIPE, 16), jnp.float32)
    zeros128 = jnp.zeros((STRIPE, D), jnp.float32)

    degp = _sc_degree(dst3, ones_rows, zeros16)
    hs1, dis = _tc_pre(xp, W1, degp)
    s1 = _sc_spmm(hs1, src3, dst3, zeros128)
    hs2 = _tc_mid(s1, hs1, dis, b1.reshape(1, D), W2)
    s2 = _sc_spmm(hs2, src3, dst3, zeros128)
    outp = _tc_post(s2, hs2, dis, b2.reshape(1, D),
                    Wfc, bfc.reshape(1, 1))
    return outp[:N]
